# baseline (speedup 1.0000x reference)
import functools

import jax
import jax.numpy as jnp
from jax import lax
from jax.experimental import pallas as pl
from jax.experimental.pallas import tpu as pltpu
from jax.experimental.pallas import tpu_sc as plsc

N = 10000
D = 128
E = 320000
DE = 144
NS = 16
NC = 2
CHUNK = 128
G = 4
C = 160
NG = C // G
EP = C * NS * CHUNK
CTOT = C * NS
N_ACC = 10016
DUMMY = N
R_ACC = N_ACC // NS
R_OUT = N // NS
BLK = 1000


def _hh_block(x, v_raw, a):
    nrm = jnp.sqrt(jnp.sum(v_raw * v_raw, axis=1, keepdims=True)) + 1e-6
    v = v_raw / nrm
    sp = jax.nn.softplus(a)
    return sp * (x - 2.0 * v * jnp.sum(v * x, axis=1, keepdims=True))


def _pre_body(x_r, vs_r, as_r, vt_r, at_r, g_r, h_r):
    x = x_r[...]
    g_r[...] = _hh_block(x, vs_r[...], as_r[...])
    h_r[...] = _hh_block(x, vt_r[...], at_r[...])


def _pre(x, v_src, a_src, v_tgt, a_tgt):
    nblk = N // BLK
    row = lambda i: (i, 0)
    return pl.pallas_call(
        _pre_body,
        grid=(nblk,),
        in_specs=[
            pl.BlockSpec((BLK, D), row),
            pl.BlockSpec((BLK, D), row),
            pl.BlockSpec((BLK, 1), row),
            pl.BlockSpec((BLK, D), row),
            pl.BlockSpec((BLK, 1), row),
        ],
        out_specs=[pl.BlockSpec((BLK, D), row), pl.BlockSpec((BLK, D), row)],
        out_shape=[
            jax.ShapeDtypeStruct((N, D), jnp.float32),
            jax.ShapeDtypeStruct((N, D), jnp.float32),
        ],
    )(x, v_src, a_src, v_tgt, a_tgt)


def _post_body(x_r, ao_r, co_r, ai_r, ci_r, vs_r, as_r, vt_r, at_r,
               wo_r, wi_r, wf_r, b_r, eo_r, ei_r, out_r):
    x = x_r[...]
    co = co_r[...]
    ci = ci_r[...]
    SA = _hh_block(ao_r[...], vs_r[...], as_r[...])
    TA = _hh_block(ai_r[...], vt_r[...], at_r[...])
    L_out = (co * x - SA) / jnp.maximum(co, 1.0)
    L_in = (ci * x - TA) / jnp.maximum(ci, 1.0)
    y = (x
         - eo_r[0, 0] * jnp.dot(L_out, wo_r[...], preferred_element_type=jnp.float32)
         - ei_r[0, 0] * jnp.dot(L_in, wi_r[...], preferred_element_type=jnp.float32))
    out_r[...] = jnp.maximum(
        jnp.dot(y, wf_r[...], preferred_element_type=jnp.float32) + b_r[...], 0.0)


def _post(x, ao, co, ai, ci, v_src, a_src, v_tgt, a_tgt, woT, wiT, wfT, b, eo, ei):
    nblk = N // BLK
    row = lambda i: (i, 0)
    fixed = lambda i: (0, 0)
    return pl.pallas_call(
        _post_body,
        grid=(nblk,),
        in_specs=[
            pl.BlockSpec((BLK, D), row),
            pl.BlockSpec((BLK, D), row),
            pl.BlockSpec((BLK, 1), row),
            pl.BlockSpec((BLK, D), row),
            pl.BlockSpec((BLK, 1), row),
            pl.BlockSpec((BLK, D), row),
            pl.BlockSpec((BLK, 1), row),
            pl.BlockSpec((BLK, D), row),
            pl.BlockSpec((BLK, 1), row),
            pl.BlockSpec((D, D), fixed),
            pl.BlockSpec((D, D), fixed),
            pl.BlockSpec((D, D), fixed),
            pl.BlockSpec((1, D), fixed),
            pl.BlockSpec((1, 1), fixed),
            pl.BlockSpec((1, 1), fixed),
        ],
        out_specs=pl.BlockSpec((BLK, D), row),
        out_shape=jax.ShapeDtypeStruct((N, D), jnp.float32),
    )(x, ao, co, ai, ci, v_src, a_src, v_tgt, a_tgt, woT, wiT, wfT, b, eo, ei)


def _sc_body(tab_hbm, idx_hbm, z_hbm, out_hbm,
             islot0, buf0, acc, semg0):
    c = lax.axis_index("c")
    sid = lax.axis_index("s")
    base = sid * C
    pltpu.sync_copy(z_hbm.at[pl.ds(sid * R_ACC, R_ACC)],
                    acc.at[pl.ds(sid * R_ACC, R_ACC)])
    plsc.subcore_barrier()

    idx = idx_hbm.at[c]

    def body(j, carry):
        pltpu.sync_copy(idx.at[base + j], islot0)
        pltpu.async_copy(tab_hbm.at[islot0.at[0]], buf0, semg0).wait()
        pltpu.sync_copy(buf0, acc.at[islot0.at[1]], add=True)
        return carry

    lax.fori_loop(0, C, body, 0)
    plsc.subcore_barrier()
    pltpu.sync_copy(acc.at[pl.ds(sid * R_OUT, R_OUT)],
                    out_hbm.at[c].at[pl.ds(sid * R_OUT, R_OUT)])


@functools.lru_cache(maxsize=None)
def _sc_call():
    return functools.partial(
        pl.kernel,
        mesh=plsc.VectorSubcoreMesh(core_axis_name="c", subcore_axis_name="s"),
        compiler_params=pltpu.CompilerParams(use_tc_tiling_on_sc=False),
        out_type=jax.ShapeDtypeStruct((NC, N, DE), jnp.float32),
        scratch_types=[
            pltpu.VMEM((2, CHUNK), jnp.int32),
            pltpu.VMEM((CHUNK, DE), jnp.float32),
            pltpu.VMEM_SHARED((N_ACC, DE), jnp.float32),
            pltpu.SemaphoreType.DMA,
        ],
    )(_sc_body)


def kernel(x, edge_index, v_src, v_tgt, alpha_src, alpha_tgt,
           W_out, W_in, W_feat, b_feat, eps_out, eps_in):
    a_src = alpha_src.reshape(N, 1)
    a_tgt = alpha_tgt.reshape(N, 1)
    g, h = _pre(x, v_src, a_src, v_tgt, a_tgt)

    ones = jnp.ones((N, 1), jnp.float32)
    zer = jnp.zeros((N, DE - D - 1), jnp.float32)
    tab = jnp.concatenate(
        [jnp.concatenate([g, ones, zer], axis=1),
         jnp.concatenate([h, ones, zer], axis=1)], axis=0)

    src = edge_index[0]
    dst = edge_index[1]
    pad = EP - E
    zpad = jnp.zeros((pad,), jnp.int32)
    dpad = jnp.full((pad,), DUMMY, jnp.int32)
    gidx = jnp.stack([
        jnp.concatenate([dst, zpad]),
        jnp.concatenate([src + N, zpad]),
    ]).reshape(NC, CTOT, CHUNK)
    sidx = jnp.stack([
        jnp.concatenate([src, dpad]),
        jnp.concatenate([dst, dpad]),
    ]).reshape(NC, CTOT, CHUNK)
    idx = jnp.stack([gidx, sidx], axis=2)
    zacc = jnp.zeros((N_ACC, DE), jnp.float32)

    A = _sc_call()(tab, idx, zacc)

    out = _post(x, A[0, :, :D], A[0, :, D:D + 1], A[1, :, :D], A[1, :, D:D + 1],
                v_src, a_src, v_tgt, a_tgt,
                W_out.T, W_in.T, W_feat.T, b_feat.reshape(1, D),
                eps_out.reshape(1, 1), eps_in.reshape(1, 1))
    return out


# speedup vs baseline: 1.7599x; 1.7599x over previous
import functools

import jax
import jax.numpy as jnp
from jax import lax
from jax.experimental import pallas as pl
from jax.experimental.pallas import tpu as pltpu
from jax.experimental.pallas import tpu_sc as plsc

N = 10000
D = 128
E = 320000
DE = 144
NS = 16
NC = 2
CHUNK = 128
G = 4
C = 160
NG = C // G
EP = C * NS * CHUNK
CTOT = C * NS
N_ACC = 10128
DUMMY = N
R_ACC = N_ACC // NS
R_OUT = N // NS
BLK = 1000


def _hh_block(x, v_raw, a):
    nrm = jnp.sqrt(jnp.sum(v_raw * v_raw, axis=1, keepdims=True)) + 1e-6
    v = v_raw / nrm
    sp = jax.nn.softplus(a)
    return sp * (x - 2.0 * v * jnp.sum(v * x, axis=1, keepdims=True))


def _pre_body(x_r, vs_r, as_r, vt_r, at_r, g_r, h_r):
    x = x_r[...]
    g_r[...] = _hh_block(x, vs_r[...], as_r[...])
    h_r[...] = _hh_block(x, vt_r[...], at_r[...])


def _pre(x, v_src, a_src, v_tgt, a_tgt):
    nblk = N // BLK
    row = lambda i: (i, 0)
    return pl.pallas_call(
        _pre_body,
        grid=(nblk,),
        in_specs=[
            pl.BlockSpec((BLK, D), row),
            pl.BlockSpec((BLK, D), row),
            pl.BlockSpec((BLK, 1), row),
            pl.BlockSpec((BLK, D), row),
            pl.BlockSpec((BLK, 1), row),
        ],
        out_specs=[pl.BlockSpec((BLK, D), row), pl.BlockSpec((BLK, D), row)],
        out_shape=[
            jax.ShapeDtypeStruct((N, D), jnp.float32),
            jax.ShapeDtypeStruct((N, D), jnp.float32),
        ],
    )(x, v_src, a_src, v_tgt, a_tgt)


def _post_body(x_r, ao_r, co_r, ai_r, ci_r, vs_r, as_r, vt_r, at_r,
               wo_r, wi_r, wf_r, b_r, eo_r, ei_r, out_r):
    x = x_r[...]
    co = co_r[...]
    ci = ci_r[...]
    SA = _hh_block(ao_r[...], vs_r[...], as_r[...])
    TA = _hh_block(ai_r[...], vt_r[...], at_r[...])
    L_out = (co * x - SA) / jnp.maximum(co, 1.0)
    L_in = (ci * x - TA) / jnp.maximum(ci, 1.0)
    y = (x
         - eo_r[0, 0] * jnp.dot(L_out, wo_r[...], preferred_element_type=jnp.float32)
         - ei_r[0, 0] * jnp.dot(L_in, wi_r[...], preferred_element_type=jnp.float32))
    out_r[...] = jnp.maximum(
        jnp.dot(y, wf_r[...], preferred_element_type=jnp.float32) + b_r[...], 0.0)


def _post(x, ao, co, ai, ci, v_src, a_src, v_tgt, a_tgt, woT, wiT, wfT, b, eo, ei):
    nblk = N // BLK
    row = lambda i: (i, 0)
    fixed = lambda i: (0, 0)
    return pl.pallas_call(
        _post_body,
        grid=(nblk,),
        in_specs=[
            pl.BlockSpec((BLK, D), row),
            pl.BlockSpec((BLK, D), row),
            pl.BlockSpec((BLK, 1), row),
            pl.BlockSpec((BLK, D), row),
            pl.BlockSpec((BLK, 1), row),
            pl.BlockSpec((BLK, D), row),
            pl.BlockSpec((BLK, 1), row),
            pl.BlockSpec((BLK, D), row),
            pl.BlockSpec((BLK, 1), row),
            pl.BlockSpec((D, D), fixed),
            pl.BlockSpec((D, D), fixed),
            pl.BlockSpec((D, D), fixed),
            pl.BlockSpec((1, D), fixed),
            pl.BlockSpec((1, 1), fixed),
            pl.BlockSpec((1, 1), fixed),
        ],
        out_specs=pl.BlockSpec((BLK, D), row),
        out_shape=jax.ShapeDtypeStruct((N, D), jnp.float32),
    )(x, ao, co, ai, ci, v_src, a_src, v_tgt, a_tgt, woT, wiT, wfT, b, eo, ei)


def _sc_body(tab_hbm, idx_hbm, z_hbm, out_hbm,
             islot0, buf0, acc, semg0):
    c = lax.axis_index("c")
    sid = lax.axis_index("s")
    base = sid * C
    pltpu.sync_copy(z_hbm.at[pl.ds(sid * R_ACC, R_ACC)],
                    acc.at[pl.ds(sid * R_ACC, R_ACC)])
    plsc.subcore_barrier()

    idx = idx_hbm.at[c]

    def body(j, carry):
        pltpu.sync_copy(idx.at[base + j], islot0)
        pltpu.async_copy(tab_hbm.at[islot0.at[0]], buf0, semg0).wait()
        pltpu.sync_copy(buf0, acc.at[islot0.at[1]], add=True)
        return carry

    lax.fori_loop(0, C, body, 0)
    plsc.subcore_barrier()
    pltpu.sync_copy(acc.at[pl.ds(sid * R_OUT, R_OUT)],
                    out_hbm.at[c].at[pl.ds(sid * R_OUT, R_OUT)])


@functools.lru_cache(maxsize=None)
def _sc_call():
    return functools.partial(
        pl.kernel,
        mesh=plsc.VectorSubcoreMesh(core_axis_name="c", subcore_axis_name="s"),
        compiler_params=pltpu.CompilerParams(use_tc_tiling_on_sc=False),
        out_type=jax.ShapeDtypeStruct((NC, N, DE), jnp.float32),
        scratch_types=[
            pltpu.VMEM((2, CHUNK), jnp.int32),
            pltpu.VMEM((CHUNK, DE), jnp.float32),
            pltpu.VMEM_SHARED((N_ACC, DE), jnp.float32),
            pltpu.SemaphoreType.DMA,
        ],
    )(_sc_body)


def kernel(x, edge_index, v_src, v_tgt, alpha_src, alpha_tgt,
           W_out, W_in, W_feat, b_feat, eps_out, eps_in):
    a_src = alpha_src.reshape(N, 1)
    a_tgt = alpha_tgt.reshape(N, 1)
    g, h = _pre(x, v_src, a_src, v_tgt, a_tgt)

    ones = jnp.ones((N, 1), jnp.float32)
    zer = jnp.zeros((N, DE - D - 1), jnp.float32)
    tab = jnp.concatenate(
        [jnp.concatenate([g, ones, zer], axis=1),
         jnp.concatenate([h, ones, zer], axis=1)], axis=0)

    src = edge_index[0]
    dst = edge_index[1]
    pad = EP - E
    zpad = (jnp.arange(pad, dtype=jnp.int32) * 61) % N
    dpad = DUMMY + (jnp.arange(pad, dtype=jnp.int32) % (N_ACC - N))
    gidx = jnp.stack([
        jnp.concatenate([dst, zpad]),
        jnp.concatenate([src + N, zpad]),
    ]).reshape(NC, CTOT, CHUNK)
    sidx = jnp.stack([
        jnp.concatenate([src, dpad]),
        jnp.concatenate([dst, dpad]),
    ]).reshape(NC, CTOT, CHUNK)
    idx = jnp.stack([gidx, sidx], axis=2)
    zacc = jnp.zeros((N_ACC, DE), jnp.float32)

    A = _sc_call()(tab, idx, zacc)

    out = _post(x, A[0, :, :D], A[0, :, D:D + 1], A[1, :, :D], A[1, :, D:D + 1],
                v_src, a_src, v_tgt, a_tgt,
                W_out.T, W_in.T, W_feat.T, b_feat.reshape(1, D),
                eps_out.reshape(1, 1), eps_in.reshape(1, 1))
    return out


# speedup vs baseline: 2.4719x; 1.4046x over previous
import functools

import jax
import jax.numpy as jnp
from jax import lax
from jax.experimental import pallas as pl
from jax.experimental.pallas import tpu as pltpu
from jax.experimental.pallas import tpu_sc as plsc

N = 10000
D = 128
E = 320000
DE = 144
NS = 16
NC = 2
CHUNK = 128
G = 4
C = 160
NG = C // G
EP = C * NS * CHUNK
CTOT = C * NS
N_ACC = 10128
DUMMY = N
R_ACC = N_ACC // NS
R_OUT = N // NS
BLK = 1000


def _hh_block(x, v_raw, a):
    nrm = jnp.sqrt(jnp.sum(v_raw * v_raw, axis=1, keepdims=True)) + 1e-6
    v = v_raw / nrm
    sp = jax.nn.softplus(a)
    return sp * (x - 2.0 * v * jnp.sum(v * x, axis=1, keepdims=True))


def _pre_body(x_r, vs_r, as_r, vt_r, at_r, g_r, h_r):
    x = x_r[...]
    g_r[...] = _hh_block(x, vs_r[...], as_r[...])
    h_r[...] = _hh_block(x, vt_r[...], at_r[...])


def _pre(x, v_src, a_src, v_tgt, a_tgt):
    nblk = N // BLK
    row = lambda i: (i, 0)
    return pl.pallas_call(
        _pre_body,
        grid=(nblk,),
        in_specs=[
            pl.BlockSpec((BLK, D), row),
            pl.BlockSpec((BLK, D), row),
            pl.BlockSpec((BLK, 1), row),
            pl.BlockSpec((BLK, D), row),
            pl.BlockSpec((BLK, 1), row),
        ],
        out_specs=[pl.BlockSpec((BLK, D), row), pl.BlockSpec((BLK, D), row)],
        out_shape=[
            jax.ShapeDtypeStruct((N, D), jnp.float32),
            jax.ShapeDtypeStruct((N, D), jnp.float32),
        ],
    )(x, v_src, a_src, v_tgt, a_tgt)


def _post_body(x_r, ao_r, co_r, ai_r, ci_r, vs_r, as_r, vt_r, at_r,
               wo_r, wi_r, wf_r, b_r, eo_r, ei_r, out_r):
    x = x_r[...]
    co = co_r[...]
    ci = ci_r[...]
    SA = _hh_block(ao_r[...], vs_r[...], as_r[...])
    TA = _hh_block(ai_r[...], vt_r[...], at_r[...])
    L_out = (co * x - SA) / jnp.maximum(co, 1.0)
    L_in = (ci * x - TA) / jnp.maximum(ci, 1.0)
    y = (x
         - eo_r[0, 0] * jnp.dot(L_out, wo_r[...], preferred_element_type=jnp.float32)
         - ei_r[0, 0] * jnp.dot(L_in, wi_r[...], preferred_element_type=jnp.float32))
    out_r[...] = jnp.maximum(
        jnp.dot(y, wf_r[...], preferred_element_type=jnp.float32) + b_r[...], 0.0)


def _post(x, ao, co, ai, ci, v_src, a_src, v_tgt, a_tgt, woT, wiT, wfT, b, eo, ei):
    nblk = N // BLK
    row = lambda i: (i, 0)
    fixed = lambda i: (0, 0)
    return pl.pallas_call(
        _post_body,
        grid=(nblk,),
        in_specs=[
            pl.BlockSpec((BLK, D), row),
            pl.BlockSpec((BLK, D), row),
            pl.BlockSpec((BLK, 1), row),
            pl.BlockSpec((BLK, D), row),
            pl.BlockSpec((BLK, 1), row),
            pl.BlockSpec((BLK, D), row),
            pl.BlockSpec((BLK, 1), row),
            pl.BlockSpec((BLK, D), row),
            pl.BlockSpec((BLK, 1), row),
            pl.BlockSpec((D, D), fixed),
            pl.BlockSpec((D, D), fixed),
            pl.BlockSpec((D, D), fixed),
            pl.BlockSpec((1, D), fixed),
            pl.BlockSpec((1, 1), fixed),
            pl.BlockSpec((1, 1), fixed),
        ],
        out_specs=pl.BlockSpec((BLK, D), row),
        out_shape=jax.ShapeDtypeStruct((N, D), jnp.float32),
    )(x, ao, co, ai, ci, v_src, a_src, v_tgt, a_tgt, woT, wiT, wfT, b, eo, ei)


def _sc_body(tab_hbm, idx_hbm, z_hbm, out_hbm,
             islot0, islot1, buf0, buf1, dummy_v, acc,
             semi0, semi1, semg0, semg1, sems0, sems1):
    c = lax.axis_index("c")
    sid = lax.axis_index("s")
    base = sid * C
    pltpu.sync_copy(z_hbm.at[pl.ds(sid * R_ACC, R_ACC)],
                    acc.at[pl.ds(sid * R_ACC, R_ACC)])
    for k in range(CHUNK // 16):
        dummy_v[pl.ds(16 * k, 16)] = DUMMY + 16 * k + lax.iota(jnp.int32, 16)
    plsc.subcore_barrier()

    idx = idx_hbm.at[c]
    bufs = (buf0, buf1)
    semg = (semg0, semg1)
    sems = (sems0, sems1)

    def drain(dst_ref, sem):
        pltpu.make_async_copy(tab_hbm.at[pl.ds(0, CHUNK)], dst_ref, sem).wait()

    pltpu.async_copy(idx.at[pl.ds(base, G)], islot0, semi0).wait()
    pltpu.async_copy(idx.at[pl.ds(base + G, G)], islot1, semi1)
    pltpu.async_copy(buf1, acc.at[dummy_v], sems1, add=True)
    pltpu.async_copy(tab_hbm.at[islot0.at[0, 0]], buf0, semg0)

    def group(g, slot_cur, slot_nxt, semi_cur, semi_nxt, last):
        for k in range(G):
            p = k % 2
            drain(bufs[p], semg[p])
            pltpu.async_copy(bufs[p], acc.at[slot_cur.at[k, 1]],
                             sems[p], add=True)
            drain(bufs[1 - p], sems[1 - p])
            if k < G - 1:
                pltpu.async_copy(tab_hbm.at[slot_cur.at[k + 1, 0]],
                                 bufs[1 - p], semg[1 - p])
            else:
                def next_head():
                    pltpu.make_async_copy(idx.at[pl.ds(0, G)],
                                          slot_nxt, semi_nxt).wait()
                    pltpu.async_copy(tab_hbm.at[slot_nxt.at[0, 0]],
                                     bufs[1 - p], semg[1 - p])
                if last is None:
                    next_head()
                else:
                    pl.when(jnp.logical_not(last))(next_head)
        @pl.when(g + 2 < NG)
        def _():
            pltpu.async_copy(idx.at[pl.ds(base + (g + 2) * G, G)],
                             slot_cur, semi_cur)

    def body(i, carry):
        g0 = 2 * i
        group(g0, islot0, islot1, semi0, semi1, None)
        group(g0 + 1, islot1, islot0, semi1, semi0, g0 + 1 == NG - 1)
        return carry

    lax.fori_loop(0, NG // 2, body, 0)
    drain(bufs[(C - 1) % 2], sems[(C - 1) % 2])
    plsc.subcore_barrier()
    pltpu.sync_copy(acc.at[pl.ds(sid * R_OUT, R_OUT)],
                    out_hbm.at[c].at[pl.ds(sid * R_OUT, R_OUT)])


@functools.lru_cache(maxsize=None)
def _sc_call():
    return functools.partial(
        pl.kernel,
        mesh=plsc.VectorSubcoreMesh(core_axis_name="c", subcore_axis_name="s"),
        compiler_params=pltpu.CompilerParams(use_tc_tiling_on_sc=False),
        out_type=jax.ShapeDtypeStruct((NC, N, DE), jnp.float32),
        scratch_types=[
            pltpu.VMEM((G, 2, CHUNK), jnp.int32),
            pltpu.VMEM((G, 2, CHUNK), jnp.int32),
            pltpu.VMEM((CHUNK, DE), jnp.float32),
            pltpu.VMEM((CHUNK, DE), jnp.float32),
            pltpu.VMEM((CHUNK,), jnp.int32),
            pltpu.VMEM_SHARED((N_ACC, DE), jnp.float32),
            pltpu.SemaphoreType.DMA,
            pltpu.SemaphoreType.DMA,
            pltpu.SemaphoreType.DMA,
            pltpu.SemaphoreType.DMA,
            pltpu.SemaphoreType.DMA,
            pltpu.SemaphoreType.DMA,
        ],
    )(_sc_body)


def kernel(x, edge_index, v_src, v_tgt, alpha_src, alpha_tgt,
           W_out, W_in, W_feat, b_feat, eps_out, eps_in):
    a_src = alpha_src.reshape(N, 1)
    a_tgt = alpha_tgt.reshape(N, 1)
    g, h = _pre(x, v_src, a_src, v_tgt, a_tgt)

    ones = jnp.ones((N, 1), jnp.float32)
    zer = jnp.zeros((N, DE - D - 1), jnp.float32)
    tab = jnp.concatenate(
        [jnp.concatenate([g, ones, zer], axis=1),
         jnp.concatenate([h, ones, zer], axis=1)], axis=0)

    src = edge_index[0]
    dst = edge_index[1]
    pad = EP - E
    zpad = (jnp.arange(pad, dtype=jnp.int32) * 61) % N
    dpad = DUMMY + (jnp.arange(pad, dtype=jnp.int32) % (N_ACC - N))
    gidx = jnp.stack([
        jnp.concatenate([dst, zpad]),
        jnp.concatenate([src + N, zpad]),
    ]).reshape(NC, CTOT, CHUNK)
    sidx = jnp.stack([
        jnp.concatenate([src, dpad]),
        jnp.concatenate([dst, dpad]),
    ]).reshape(NC, CTOT, CHUNK)
    idx = jnp.stack([gidx, sidx], axis=2)
    zacc = jnp.zeros((N_ACC, DE), jnp.float32)

    A = _sc_call()(tab, idx, zacc)

    out = _post(x, A[0, :, :D], A[0, :, D:D + 1], A[1, :, :D], A[1, :, D:D + 1],
                v_src, a_src, v_tgt, a_tgt,
                W_out.T, W_in.T, W_feat.T, b_feat.reshape(1, D),
                eps_out.reshape(1, 1), eps_in.reshape(1, 1))
    return out


# speedup vs baseline: 2.7893x; 1.1284x over previous
import functools

import jax
import jax.numpy as jnp
from jax import lax
from jax.experimental import pallas as pl
from jax.experimental.pallas import tpu as pltpu
from jax.experimental.pallas import tpu_sc as plsc

N = 10000
D = 128
E = 320000
DE = 144
NS = 16
NC = 2
CHUNK = 128
G = 4
C = 160
NG = C // G
EP = C * NS * CHUNK
CTOT = C * NS
N_ACC = 10128
DUMMY = N
R_ACC = N_ACC // NS
R_OUT = N // NS
BLK = 1000


def _hh_block(x, v_raw, a):
    nrm = jnp.sqrt(jnp.sum(v_raw * v_raw, axis=1, keepdims=True)) + 1e-6
    v = v_raw / nrm
    sp = jax.nn.softplus(a)
    return sp * (x - 2.0 * v * jnp.sum(v * x, axis=1, keepdims=True))


def _pre_body(x_r, vs_r, as_r, vt_r, at_r, tab_r):
    x = x_r[...]
    ones = jnp.ones((BLK, 1), jnp.float32)
    zer = jnp.zeros((BLK, DE - D - 1), jnp.float32)
    g = _hh_block(x, vs_r[...], as_r[...])
    h = _hh_block(x, vt_r[...], at_r[...])
    tab_r[0] = jnp.concatenate([g, ones, zer], axis=1)
    tab_r[1] = jnp.concatenate([h, ones, zer], axis=1)


def _pre(x, v_src, a_src, v_tgt, a_tgt):
    nblk = N // BLK
    row = lambda i: (i, 0)
    return pl.pallas_call(
        _pre_body,
        grid=(nblk,),
        in_specs=[
            pl.BlockSpec((BLK, D), row),
            pl.BlockSpec((BLK, D), row),
            pl.BlockSpec((BLK, 1), row),
            pl.BlockSpec((BLK, D), row),
            pl.BlockSpec((BLK, 1), row),
        ],
        out_specs=pl.BlockSpec((2, BLK, DE), lambda i: (0, i, 0)),
        out_shape=jax.ShapeDtypeStruct((2, N, DE), jnp.float32),
    )(x, v_src, a_src, v_tgt, a_tgt)


def _post_body(x_r, ao_r, ai_r, vs_r, as_r, vt_r, at_r,
               wo_r, wi_r, wf_r, b_r, eo_r, ei_r, out_r):
    x = x_r[...]
    co = ao_r[0, :, D:D + 1]
    ci = ai_r[0, :, D:D + 1]
    SA = _hh_block(ao_r[0, :, :D], vs_r[...], as_r[...])
    TA = _hh_block(ai_r[0, :, :D], vt_r[...], at_r[...])
    L_out = (co * x - SA) / jnp.maximum(co, 1.0)
    L_in = (ci * x - TA) / jnp.maximum(ci, 1.0)
    y = (x
         - eo_r[0, 0] * jnp.dot(L_out, wo_r[...], preferred_element_type=jnp.float32)
         - ei_r[0, 0] * jnp.dot(L_in, wi_r[...], preferred_element_type=jnp.float32))
    out_r[...] = jnp.maximum(
        jnp.dot(y, wf_r[...], preferred_element_type=jnp.float32) + b_r[...], 0.0)


def _post(x, A, v_src, a_src, v_tgt, a_tgt, woT, wiT, wfT, b, eo, ei):
    nblk = N // BLK
    row = lambda i: (i, 0)
    fixed = lambda i: (0, 0)
    return pl.pallas_call(
        _post_body,
        grid=(nblk,),
        in_specs=[
            pl.BlockSpec((BLK, D), row),
            pl.BlockSpec((1, BLK, DE), lambda i: (0, i, 0)),
            pl.BlockSpec((1, BLK, DE), lambda i: (1, i, 0)),
            pl.BlockSpec((BLK, D), row),
            pl.BlockSpec((BLK, 1), row),
            pl.BlockSpec((BLK, D), row),
            pl.BlockSpec((BLK, 1), row),
            pl.BlockSpec((D, D), fixed),
            pl.BlockSpec((D, D), fixed),
            pl.BlockSpec((D, D), fixed),
            pl.BlockSpec((1, D), fixed),
            pl.BlockSpec((1, 1), fixed),
            pl.BlockSpec((1, 1), fixed),
        ],
        out_specs=pl.BlockSpec((BLK, D), row),
        out_shape=jax.ShapeDtypeStruct((N, D), jnp.float32),
    )(x, A, A, v_src, a_src, v_tgt, a_tgt, woT, wiT, wfT, b, eo, ei)


def _sc_body(tab_hbm, idx_hbm, z_hbm, out_hbm,
             islot0, islot1, buf0, buf1, dummy_v, acc,
             semi0, semi1, semg0, semg1, sems0, sems1):
    c = lax.axis_index("c")
    sid = lax.axis_index("s")
    base = sid * C
    pltpu.sync_copy(z_hbm, acc.at[pl.ds(sid * R_ACC, R_ACC)])
    for k in range(CHUNK // 16):
        dummy_v[pl.ds(16 * k, 16)] = DUMMY + 16 * k + lax.iota(jnp.int32, 16)
    plsc.subcore_barrier()

    idx = idx_hbm.at[c]
    bufs = (buf0, buf1)
    semg = (semg0, semg1)
    sems = (sems0, sems1)

    def drain(dst_ref, sem):
        pltpu.make_async_copy(tab_hbm.at[pl.ds(0, CHUNK)], dst_ref, sem).wait()

    pltpu.async_copy(idx.at[pl.ds(base, G)], islot0, semi0).wait()
    pltpu.async_copy(idx.at[pl.ds(base + G, G)], islot1, semi1)
    pltpu.async_copy(buf1, acc.at[dummy_v], sems1, add=True)
    pltpu.async_copy(tab_hbm.at[islot0.at[0, 0]], buf0, semg0)

    def group(g, slot_cur, slot_nxt, semi_cur, semi_nxt, last):
        for k in range(G):
            p = k % 2
            drain(bufs[p], semg[p])
            pltpu.async_copy(bufs[p], acc.at[slot_cur.at[k, 1]],
                             sems[p], add=True)
            drain(bufs[1 - p], sems[1 - p])
            if k < G - 1:
                pltpu.async_copy(tab_hbm.at[slot_cur.at[k + 1, 0]],
                                 bufs[1 - p], semg[1 - p])
            else:
                def next_head():
                    pltpu.make_async_copy(idx.at[pl.ds(0, G)],
                                          slot_nxt, semi_nxt).wait()
                    pltpu.async_copy(tab_hbm.at[slot_nxt.at[0, 0]],
                                     bufs[1 - p], semg[1 - p])
                if last is None:
                    next_head()
                else:
                    pl.when(jnp.logical_not(last))(next_head)
        @pl.when(g + 2 < NG)
        def _():
            pltpu.async_copy(idx.at[pl.ds(base + (g + 2) * G, G)],
                             slot_cur, semi_cur)

    def body(i, carry):
        g0 = 2 * i
        group(g0, islot0, islot1, semi0, semi1, None)
        group(g0 + 1, islot1, islot0, semi1, semi0, g0 + 1 == NG - 1)
        return carry

    lax.fori_loop(0, NG // 2, body, 0)
    drain(bufs[(C - 1) % 2], sems[(C - 1) % 2])
    plsc.subcore_barrier()
    pltpu.sync_copy(acc.at[pl.ds(sid * R_OUT, R_OUT)],
                    out_hbm.at[c].at[pl.ds(sid * R_OUT, R_OUT)])


@functools.lru_cache(maxsize=None)
def _sc_call():
    return functools.partial(
        pl.kernel,
        mesh=plsc.VectorSubcoreMesh(core_axis_name="c", subcore_axis_name="s"),
        compiler_params=pltpu.CompilerParams(use_tc_tiling_on_sc=False),
        out_type=jax.ShapeDtypeStruct((NC, N, DE), jnp.float32),
        scratch_types=[
            pltpu.VMEM((G, 2, CHUNK), jnp.int32),
            pltpu.VMEM((G, 2, CHUNK), jnp.int32),
            pltpu.VMEM((CHUNK, DE), jnp.float32),
            pltpu.VMEM((CHUNK, DE), jnp.float32),
            pltpu.VMEM((CHUNK,), jnp.int32),
            pltpu.VMEM_SHARED((N_ACC, DE), jnp.float32),
            pltpu.SemaphoreType.DMA,
            pltpu.SemaphoreType.DMA,
            pltpu.SemaphoreType.DMA,
            pltpu.SemaphoreType.DMA,
            pltpu.SemaphoreType.DMA,
            pltpu.SemaphoreType.DMA,
        ],
    )(_sc_body)


def kernel(x, edge_index, v_src, v_tgt, alpha_src, alpha_tgt,
           W_out, W_in, W_feat, b_feat, eps_out, eps_in):
    a_src = alpha_src.reshape(N, 1)
    a_tgt = alpha_tgt.reshape(N, 1)
    tab = _pre(x, v_src, a_src, v_tgt, a_tgt).reshape(2 * N, DE)

    src = edge_index[0]
    dst = edge_index[1]
    pad = EP - E
    zpad = (jnp.arange(pad, dtype=jnp.int32) * 61) % N
    dpad = DUMMY + (jnp.arange(pad, dtype=jnp.int32) % (N_ACC - N))
    gidx = jnp.stack([
        jnp.concatenate([dst, zpad]),
        jnp.concatenate([src + N, zpad]),
    ]).reshape(NC, CTOT, CHUNK)
    sidx = jnp.stack([
        jnp.concatenate([src, dpad]),
        jnp.concatenate([dst, dpad]),
    ]).reshape(NC, CTOT, CHUNK)
    idx = jnp.stack([gidx, sidx], axis=2)
    zacc = jnp.zeros((R_ACC, DE), jnp.float32)

    A = _sc_call()(tab, idx, zacc)

    out = _post(x, A, v_src, a_src, v_tgt, a_tgt,
                W_out.T, W_in.T, W_feat.T, b_feat.reshape(1, D),
                eps_out.reshape(1, 1), eps_in.reshape(1, 1))
    return out


# speedup vs baseline: 2.8003x; 1.0039x over previous
import functools

import jax
import jax.numpy as jnp
from jax import lax
from jax.experimental import pallas as pl
from jax.experimental.pallas import tpu as pltpu
from jax.experimental.pallas import tpu_sc as plsc

N = 10000
D = 128
E = 320000
DE = 144
NS = 16
NC = 2
CHUNK = 128
G = 4
C = 160
NG = C // G
EP = C * NS * CHUNK
CTOT = C * NS
N_ACC = 10128
DUMMY = N
R_ACC = N_ACC // NS
R_OUT = N // NS
BLK = 2000


def _hh_block(x, v_raw, a):
    nrm = jnp.sqrt(jnp.sum(v_raw * v_raw, axis=1, keepdims=True)) + 1e-6
    v = v_raw / nrm
    sp = jax.nn.softplus(a)
    return sp * (x - 2.0 * v * jnp.sum(v * x, axis=1, keepdims=True))


def _pre_body(x_r, vs_r, as_r, vt_r, at_r, tab_r):
    x = x_r[...]
    ones = jnp.ones((BLK, 1), jnp.float32)
    zer = jnp.zeros((BLK, DE - D - 1), jnp.float32)
    g = _hh_block(x, vs_r[...], as_r[...])
    h = _hh_block(x, vt_r[...], at_r[...])
    tab_r[0] = jnp.concatenate([g, ones, zer], axis=1)
    tab_r[1] = jnp.concatenate([h, ones, zer], axis=1)


def _pre(x, v_src, a_src, v_tgt, a_tgt):
    nblk = N // BLK
    row = lambda i: (i, 0)
    return pl.pallas_call(
        _pre_body,
        grid=(nblk,),
        in_specs=[
            pl.BlockSpec((BLK, D), row),
            pl.BlockSpec((BLK, D), row),
            pl.BlockSpec((BLK, 1), row),
            pl.BlockSpec((BLK, D), row),
            pl.BlockSpec((BLK, 1), row),
        ],
        out_specs=pl.BlockSpec((2, BLK, DE), lambda i: (0, i, 0)),
        out_shape=jax.ShapeDtypeStruct((2, N, DE), jnp.float32),
    )(x, v_src, a_src, v_tgt, a_tgt)


def _post_body(x_r, ao_r, ai_r, vs_r, as_r, vt_r, at_r,
               wo_r, wi_r, wf_r, b_r, eo_r, ei_r, out_r):
    x = x_r[...]
    co = ao_r[0, :, D:D + 1]
    ci = ai_r[0, :, D:D + 1]
    SA = _hh_block(ao_r[0, :, :D], vs_r[...], as_r[...])
    TA = _hh_block(ai_r[0, :, :D], vt_r[...], at_r[...])
    L_out = (co * x - SA) / jnp.maximum(co, 1.0)
    L_in = (ci * x - TA) / jnp.maximum(ci, 1.0)
    y = (x
         - eo_r[0, 0] * jnp.dot(L_out, wo_r[...], preferred_element_type=jnp.float32)
         - ei_r[0, 0] * jnp.dot(L_in, wi_r[...], preferred_element_type=jnp.float32))
    out_r[...] = jnp.maximum(
        jnp.dot(y, wf_r[...], preferred_element_type=jnp.float32) + b_r[...], 0.0)


def _post(x, A, v_src, a_src, v_tgt, a_tgt, woT, wiT, wfT, b, eo, ei):
    nblk = N // BLK
    row = lambda i: (i, 0)
    fixed = lambda i: (0, 0)
    return pl.pallas_call(
        _post_body,
        grid=(nblk,),
        in_specs=[
            pl.BlockSpec((BLK, D), row),
            pl.BlockSpec((1, BLK, DE), lambda i: (0, i, 0)),
            pl.BlockSpec((1, BLK, DE), lambda i: (1, i, 0)),
            pl.BlockSpec((BLK, D), row),
            pl.BlockSpec((BLK, 1), row),
            pl.BlockSpec((BLK, D), row),
            pl.BlockSpec((BLK, 1), row),
            pl.BlockSpec((D, D), fixed),
            pl.BlockSpec((D, D), fixed),
            pl.BlockSpec((D, D), fixed),
            pl.BlockSpec((1, D), fixed),
            pl.BlockSpec((1, 1), fixed),
            pl.BlockSpec((1, 1), fixed),
        ],
        out_specs=pl.BlockSpec((BLK, D), row),
        out_shape=jax.ShapeDtypeStruct((N, D), jnp.float32),
    )(x, A, A, v_src, a_src, v_tgt, a_tgt, woT, wiT, wfT, b, eo, ei)


def _sc_body(tab_hbm, idx_hbm, z_hbm, out_hbm,
             islot0, islot1, buf0, buf1, dummy_v, acc,
             semi0, semi1, semg0, semg1, sems0, sems1):
    c = lax.axis_index("c")
    sid = lax.axis_index("s")
    base = sid * C
    pltpu.sync_copy(z_hbm, acc.at[pl.ds(sid * R_ACC, R_ACC)])
    for k in range(CHUNK // 16):
        dummy_v[pl.ds(16 * k, 16)] = DUMMY + 16 * k + lax.iota(jnp.int32, 16)
    plsc.subcore_barrier()

    idx = idx_hbm.at[c]
    bufs = (buf0, buf1)
    semg = (semg0, semg1)
    sems = (sems0, sems1)

    def drain(dst_ref, sem):
        pltpu.make_async_copy(tab_hbm.at[pl.ds(0, CHUNK)], dst_ref, sem).wait()

    pltpu.async_copy(idx.at[pl.ds(base, G)], islot0, semi0).wait()
    pltpu.async_copy(idx.at[pl.ds(base + G, G)], islot1, semi1)
    pltpu.async_copy(buf1, acc.at[dummy_v], sems1, add=True)
    pltpu.async_copy(tab_hbm.at[islot0.at[0, 0]], buf0, semg0)

    def group(g, slot_cur, slot_nxt, semi_cur, semi_nxt, last):
        for k in range(G):
            p = k % 2
            drain(bufs[p], semg[p])
            pltpu.async_copy(bufs[p], acc.at[slot_cur.at[k, 1]],
                             sems[p], add=True)
            drain(bufs[1 - p], sems[1 - p])
            if k < G - 1:
                pltpu.async_copy(tab_hbm.at[slot_cur.at[k + 1, 0]],
                                 bufs[1 - p], semg[1 - p])
            else:
                def next_head():
                    pltpu.make_async_copy(idx.at[pl.ds(0, G)],
                                          slot_nxt, semi_nxt).wait()
                    pltpu.async_copy(tab_hbm.at[slot_nxt.at[0, 0]],
                                     bufs[1 - p], semg[1 - p])
                if last is None:
                    next_head()
                else:
                    pl.when(jnp.logical_not(last))(next_head)
        @pl.when(g + 2 < NG)
        def _():
            pltpu.async_copy(idx.at[pl.ds(base + (g + 2) * G, G)],
                             slot_cur, semi_cur)

    def body(i, carry):
        g0 = 2 * i
        group(g0, islot0, islot1, semi0, semi1, None)
        group(g0 + 1, islot1, islot0, semi1, semi0, g0 + 1 == NG - 1)
        return carry

    lax.fori_loop(0, NG // 2, body, 0)
    drain(bufs[(C - 1) % 2], sems[(C - 1) % 2])
    plsc.subcore_barrier()
    pltpu.sync_copy(acc.at[pl.ds(sid * R_OUT, R_OUT)],
                    out_hbm.at[c].at[pl.ds(sid * R_OUT, R_OUT)])


@functools.lru_cache(maxsize=None)
def _sc_call():
    return functools.partial(
        pl.kernel,
        mesh=plsc.VectorSubcoreMesh(core_axis_name="c", subcore_axis_name="s"),
        compiler_params=pltpu.CompilerParams(use_tc_tiling_on_sc=False),
        out_type=jax.ShapeDtypeStruct((NC, N, DE), jnp.float32),
        scratch_types=[
            pltpu.VMEM((G, 2, CHUNK), jnp.int32),
            pltpu.VMEM((G, 2, CHUNK), jnp.int32),
            pltpu.VMEM((CHUNK, DE), jnp.float32),
            pltpu.VMEM((CHUNK, DE), jnp.float32),
            pltpu.VMEM((CHUNK,), jnp.int32),
            pltpu.VMEM_SHARED((N_ACC, DE), jnp.float32),
            pltpu.SemaphoreType.DMA,
            pltpu.SemaphoreType.DMA,
            pltpu.SemaphoreType.DMA,
            pltpu.SemaphoreType.DMA,
            pltpu.SemaphoreType.DMA,
            pltpu.SemaphoreType.DMA,
        ],
    )(_sc_body)


def kernel(x, edge_index, v_src, v_tgt, alpha_src, alpha_tgt,
           W_out, W_in, W_feat, b_feat, eps_out, eps_in):
    a_src = alpha_src.reshape(N, 1)
    a_tgt = alpha_tgt.reshape(N, 1)
    tab = _pre(x, v_src, a_src, v_tgt, a_tgt).reshape(2 * N, DE)

    src = edge_index[0]
    dst = edge_index[1]
    pad = EP - E
    zpad = (jnp.arange(pad, dtype=jnp.int32) * 61) % N
    dpad = DUMMY + (jnp.arange(pad, dtype=jnp.int32) % (N_ACC - N))
    gidx = jnp.stack([
        jnp.concatenate([dst, zpad]),
        jnp.concatenate([src + N, zpad]),
    ]).reshape(NC, CTOT, CHUNK)
    sidx = jnp.stack([
        jnp.concatenate([src, dpad]),
        jnp.concatenate([dst, dpad]),
    ]).reshape(NC, CTOT, CHUNK)
    idx = jnp.stack([gidx, sidx], axis=2)
    zacc = jnp.zeros((R_ACC, DE), jnp.float32)

    A = _sc_call()(tab, idx, zacc)

    out = _post(x, A, v_src, a_src, v_tgt, a_tgt,
                W_out.T, W_in.T, W_feat.T, b_feat.reshape(1, D),
                eps_out.reshape(1, 1), eps_in.reshape(1, 1))
    return out
